# baseline (speedup 1.0000x reference)
import functools

import jax
import jax.numpy as jnp
from jax import lax
from jax.experimental import pallas as pl
from jax.experimental.pallas import tpu as pltpu
from jax.experimental.pallas import tpu_sc as plsc

_N = 100000
_E = 1600000
_NP = 100352
_NR = 784
_EP = 1605632
_ER = 12544
_RANGE = 25088
_ACC_ROWS = 25216
_W = 64
_F32 = jnp.float32
_I32 = jnp.int32

_MESH = plsc.VectorSubcoreMesh(core_axis_name="c", subcore_axis_name="s")
_HIGH = lax.Precision.HIGHEST


def _mm(a, b):
    return jnp.dot(a, b, precision=_HIGH)


def _deg_body(dst2d, degp, dspm, dvw, zb, ones_v):
    c = lax.axis_index("c")
    t = lax.axis_index("s")

    def zf(i, _):
        zb[pl.ds(i * 16, 16)] = jnp.zeros((16,), _F32)
        return 0
    lax.fori_loop(0, 392, zf, 0)

    def of(i, _):
        ones_v[pl.ds(i * 16, 16)] = jnp.ones((16,), _F32)
        return 0
    lax.fori_loop(0, 8, of, 0)

    pltpu.sync_copy(zb, dspm.at[pl.ds(t * 6272, 6272)])
    plsc.subcore_barrier()

    row0 = (c * 16 + t) * 392

    def wbody(w, _):
        pltpu.sync_copy(dst2d.at[pl.ds(row0 + w * 8, 8)], dvw)
        for j in range(8):
            pltpu.sync_copy(ones_v.at[pl.ds(0, 128)], dspm.at[dvw.at[j]], add=True)
        return 0
    lax.fori_loop(0, 49, wbody, 0)
    plsc.subcore_barrier()

    pltpu.sync_copy(dspm.at[pl.ds(t * 6272, 6272)], zb.at[pl.ds(0, 6272)])
    pltpu.sync_copy(zb.at[pl.ds(0, 6272)], degp.at[c, pl.ds(t * 6272, 6272)])


_deg_kernel = functools.partial(
    pl.kernel,
    out_type=jax.ShapeDtypeStruct((2, _NP), _F32),
    mesh=_MESH,
    scratch_types=[
        pltpu.VMEM_SHARED((_NP,), _F32),
        pltpu.VMEM((8, 128), _I32),
        pltpu.VMEM((6272,), _F32),
        pltpu.VMEM((128,), _F32),
    ],
)(_deg_body)


def _agg_body(src2d, dst2d, y_hbm, agg, acc, svw, dvw, fsrc, fldst, rows0,
              rows1, rows2, sem0, sem1, sem2):
    c = lax.axis_index("c")
    t = lax.axis_index("s")

    lane = lax.broadcasted_iota(_I32, (16,), 0)

    def _issue(g):
        r = g & 7
        b = g % 3
        for bi, (rw, sm) in enumerate(((rows0, sem0), (rows1, sem1),
                                       (rows2, sem2))):
            @pl.when(b == bi)
            def _(rw=rw, sm=sm):
                pltpu.async_copy(y_hbm.at[fsrc.at[r]], rw, sm)

    def _wait_scatter(d):
        r = d & 7
        b = d % 3
        for bi, (rw, sm) in enumerate(((rows0, sem0), (rows1, sem1),
                                       (rows2, sem2))):
            @pl.when(b == bi)
            def _(rw=rw, sm=sm):
                pltpu.make_async_copy(y_hbm.at[fsrc.at[r]], rw, sm).wait()
                pltpu.sync_copy(rw, acc.at[fldst.at[r]], add=True)

    for p in range(2):
        base = (2 * p + c) * _RANGE

        def zf(i, _):
            r = i // 4
            k = i % 4
            rows0[r, pl.ds(k * 16, 16)] = jnp.zeros((16,), _F32)
            return 0
        lax.fori_loop(0, 512, zf, 0)

        def zacc(k, _):
            blk = t + 16 * k

            @pl.when(blk < 197)
            def _():
                pltpu.sync_copy(rows0, acc.at[pl.ds(blk * 128, 128)])
            return 0
        lax.fori_loop(0, 13, zacc, 0)
        plsc.subcore_barrier()

        def wbody(w, carry):
            cnt, gih, drn = carry
            row0 = t * 784 + w * 8
            pltpu.sync_copy(src2d.at[pl.ds(row0, 8)], svw)
            pltpu.sync_copy(dst2d.at[pl.ds(row0, 8)], dvw)
            for j in range(64):
                jr, jc = j // 8, (j % 8) * 16
                sv = svw[jr, pl.ds(jc, 16)]
                dv = dvw[jr, pl.ds(jc, 16)]
                m = (dv >= base) & (dv < base + _RANGE)
                mi = m.astype(_I32)
                pos = cnt + plsc.cumsum(mi) - mi
                ridx = lax.shift_right_logical(pos, 7) & 7
                cidx = pos & 127
                plsc.store_scatter(fsrc, [ridx, cidx], sv, mask=m)
                plsc.store_scatter(fldst, [ridx, cidx], dv - base, mask=m)
                cnt = cnt + plsc.all_reduce_population_count(m)[0]

            def pcond(st):
                cnt_, gih_, drn_ = st
                return (gih_ + 1) * 128 <= cnt_

            def pbody(st):
                cnt_, gih_, drn_ = st
                full = gih_ - drn_ >= 3

                @pl.when(full)
                def _():
                    _wait_scatter(drn_)
                drn_ = jnp.where(full, drn_ + 1, drn_)
                _issue(gih_)
                return (cnt_, gih_ + 1, drn_)

            return lax.while_loop(pcond, pbody, (cnt, gih, drn))

        cnt, gih, drn = lax.fori_loop(
            0, 98, wbody, (jnp.int32(0), jnp.int32(0), jnp.int32(0)))

        rem = cnt & 127

        @pl.when(rem > 0)
        def _():
            r = lax.shift_right_logical(cnt, 7) & 7
            rv = jnp.broadcast_to(r, (16,))
            for k in range(8):
                p128 = lane + k * 16
                mpad = p128 >= rem
                plsc.store_scatter(fsrc, [rv, p128], t * 128 + p128, mask=mpad)
                plsc.store_scatter(fldst, [rv, p128], _RANGE + p128, mask=mpad)

        def fcond(st):
            gih_, drn_ = st
            return drn_ < gih_

        def fbody(st):
            gih_, drn_ = st
            _wait_scatter(drn_)
            return (gih_, drn_ + 1)

        gih, drn = lax.while_loop(fcond, fbody, (gih, drn))

        @pl.when(rem > 0)
        def _():
            g = lax.shift_right_logical(cnt, 7)
            _issue(g)
            _wait_scatter(g)

        plsc.subcore_barrier()
        pltpu.sync_copy(acc.at[pl.ds(t * 1568, 1568)],
                        agg.at[pl.ds(base + t * 1568, 1568)])
        plsc.subcore_barrier()


_agg_kernel = functools.partial(
    pl.kernel,
    out_type=jax.ShapeDtypeStruct((_NP, _W), _F32),
    mesh=_MESH,
    compiler_params=pltpu.CompilerParams(needs_layout_passes=False, use_tc_tiling_on_sc=False),
    scratch_types=[
        pltpu.VMEM_SHARED((_ACC_ROWS, _W), _F32),
        pltpu.VMEM((8, 128), _I32),
        pltpu.VMEM((8, 128), _I32),
        pltpu.VMEM((8, 128), _I32),
        pltpu.VMEM((8, 128), _I32),
        pltpu.VMEM((128, _W), _F32),
        pltpu.VMEM((128, _W), _F32),
        pltpu.VMEM((128, _W), _F32),
        pltpu.SemaphoreType.DMA,
        pltpu.SemaphoreType.DMA,
        pltpu.SemaphoreType.DMA,
    ],
)(_agg_body)


def _pool_body(scaled, a_hbm, y_hbm, dinvw, batch2d, pooled, accp, av, yv,
               dvv, bvv):
    c = lax.axis_index("c")
    t = lax.axis_index("s")
    w = c * 16 + t

    def inf_f(i, _):
        r = i // 4
        k = i % 4
        accp[r, pl.ds(k * 16, 16)] = jnp.full((16,), -jnp.inf, _F32)
        return 0
    lax.fori_loop(0, 260, inf_f, 0)

    def gbody(gi, _):
        g = w + 32 * gi

        @pl.when(g < _NR)
        def _():
            pltpu.sync_copy(a_hbm.at[pl.ds(g * 128, 128)], av)
            if scaled:
                pltpu.sync_copy(y_hbm.at[pl.ds(g * 128, 128)], yv)
                pltpu.sync_copy(dinvw.at[g], dvv)
            pltpu.sync_copy(batch2d.at[g], bvv)

            def rbody(rc, _):
                b16 = bvv[pl.ds(rc * 16, 16)]
                if scaled:
                    d16 = dvv[pl.ds(rc * 16, 16)]
                for k16 in range(16):
                    r = rc * 16 + k16
                    b = b16[k16]
                    for k in range(4):
                        v = av[r, pl.ds(k * 16, 16)]
                        if scaled:
                            v = (v + yv[r, pl.ds(k * 16, 16)]) * d16[k16]
                        accp[b, pl.ds(k * 16, 16)] = jnp.maximum(
                            accp[b, pl.ds(k * 16, 16)], v)
                return 0
            lax.fori_loop(0, 8, rbody, 0)
        return 0
    lax.fori_loop(0, 25, gbody, 0)

    pltpu.sync_copy(accp, pooled.at[w])


def _make_pool(scaled):
    scratch = [
        pltpu.VMEM((65, _W), _F32),
        pltpu.VMEM((128, _W), _F32),
        pltpu.VMEM((128, _W), _F32),
        pltpu.VMEM((128,), _F32),
        pltpu.VMEM((128,), _I32),
    ]
    return functools.partial(
        pl.kernel,
        out_type=jax.ShapeDtypeStruct((32, 65, _W), _F32),
        mesh=_MESH,
        compiler_params=pltpu.CompilerParams(use_tc_tiling_on_sc=False),
        scratch_types=scratch,
    )(functools.partial(_pool_body, scaled))


_pool_scaled = _make_pool(True)
_pool_plain = _make_pool(False)


def _dinv_body(degp_ref, dinv_ref):
    s = degp_ref[0:1, :] + degp_ref[1:2, :] + 1.0
    dinv_ref[...] = lax.rsqrt(s).reshape(_NR, 128)


def _dinv_tc(degp):
    return pl.pallas_call(
        _dinv_body,
        out_shape=jax.ShapeDtypeStruct((_NR, 128), _F32),
    )(degp)


def _pre_body(domx_ref, logsx_ref, dinv_ref, wl1_ref, bl1_ref, wl2_ref,
              y1_ref, mlp_ref):
    di = dinv_ref[...].reshape(8, 128, 1)
    x3 = domx_ref[...].reshape(8, 128, 50)
    y1_ref[:, 0:50] = (x3 * di).reshape(1024, 50)
    y1_ref[:, 50:64] = jnp.zeros((1024, 14), _F32)
    h = jnp.maximum(_mm(logsx_ref[...], wl1_ref[...]) + bl1_ref[...], 0.0)
    mlp_ref[...] = _mm(h, wl2_ref[...])


def _pre_tc(domx, logsx, dinvw, wl1, bl1, wl2):
    nblk = _NP // 1024
    return pl.pallas_call(
        _pre_body,
        grid=(nblk,),
        in_specs=[
            pl.BlockSpec((1024, 50), lambda i: (i, 0)),
            pl.BlockSpec((1024, 50), lambda i: (i, 0)),
            pl.BlockSpec((8, 128), lambda i: (i, 0)),
            pl.BlockSpec((50, 50), lambda i: (0, 0)),
            pl.BlockSpec((1, 50), lambda i: (0, 0)),
            pl.BlockSpec((50, 64), lambda i: (0, 0)),
        ],
        out_specs=[
            pl.BlockSpec((1024, 64), lambda i: (i, 0)),
            pl.BlockSpec((1024, 64), lambda i: (i, 0)),
        ],
        out_shape=[
            jax.ShapeDtypeStruct((_NP, 64), _F32),
            jax.ShapeDtypeStruct((_NP, 64), _F32),
        ],
    )(domx, logsx, dinvw, wl1, bl1, wl2)


def _mid_body(agg1_ref, y1_ref, dinv_ref, wg1_ref, bg1_ref, wt1_ref, bt1_ref,
              w2c_ref, y2_ref):
    di = dinv_ref[...].reshape(8, 128, 1)
    z1 = ((agg1_ref[...] + y1_ref[...]).reshape(8, 128, 64) * di).reshape(1024, 64)
    hg = jnp.maximum(_mm(z1, wg1_ref[...]) + bg1_ref[...], 0.0)
    ht = jnp.maximum(_mm(z1, wt1_ref[...]) + bt1_ref[...], 0.0)
    q = _mm(jnp.concatenate([hg, ht], axis=1), w2c_ref[...])
    y2_ref[...] = (q.reshape(8, 128, 64) * di).reshape(1024, 64)


def _mid_tc(agg1, y1p, dinvw, wg1, bg1, wt1, bt1, w2c):
    nblk = _NP // 1024
    return pl.pallas_call(
        _mid_body,
        grid=(nblk,),
        in_specs=[
            pl.BlockSpec((1024, 64), lambda i: (i, 0)),
            pl.BlockSpec((1024, 64), lambda i: (i, 0)),
            pl.BlockSpec((8, 128), lambda i: (i, 0)),
            pl.BlockSpec((64, 64), lambda i: (0, 0)),
            pl.BlockSpec((1, 64), lambda i: (0, 0)),
            pl.BlockSpec((64, 64), lambda i: (0, 0)),
            pl.BlockSpec((1, 64), lambda i: (0, 0)),
            pl.BlockSpec((128, 64), lambda i: (0, 0)),
        ],
        out_specs=pl.BlockSpec((1024, 64), lambda i: (i, 0)),
        out_shape=jax.ShapeDtypeStruct((_NP, 64), _F32),
    )(agg1, y1p, dinvw, wg1, bg1, wt1, bt1, w2c)


def _final_body(dp_ref, lp_ref, bg2_ref, bt2_ref, blg2_ref, blt2_ref, out_ref):
    dp = jnp.max(dp_ref[...], axis=0)
    lp = jnp.max(lp_ref[...], axis=0)
    gd = dp[0:64, 0:32] + bg2_ref[...]
    ad = dp[0:64, 32:64] + bt2_ref[...]
    gl = lp[0:64, 0:32] + blg2_ref[...]
    al = lp[0:64, 32:64] + blt2_ref[...]
    d1 = ad - gd + 1e-6
    d2 = al - gl + 1e-6
    out_ref[...] = (jnp.sqrt(jnp.sum(d1 * d1, axis=1, keepdims=True))
                    + jnp.sqrt(jnp.sum(d2 * d2, axis=1, keepdims=True)))


def _final_tc(dpool, lpool, bg2, bt2, blg2, blt2):
    return pl.pallas_call(
        _final_body,
        out_shape=jax.ShapeDtypeStruct((64, 1), _F32),
    )(dpool, lpool, bg2, bt2, blg2, blt2)


def kernel(dom_x, dom_edge_index, dom_batch, logs_x, logs_batch, Wg1, bg1, Wg2,
           bg2, Wt1, bt1, Wt2, bt2, Wlg1, blg1, Wlg2, blg2, Wlt1, blt1, Wlt2,
           blt2):
    dom_xp = jnp.pad(dom_x, ((0, _NP - _N), (0, 0)))
    logs_xp = jnp.pad(logs_x, ((0, _NP - _N), (0, 0)))
    npad = _EP - _E
    pidx = lax.iota(_I32, npad)
    pad_src = (pidx * 17) % _N
    pad_dst = _N + (pidx % (_NP - _N))
    src2d = jnp.concatenate([dom_edge_index[0], pad_src]).reshape(_ER, 128)
    dst2d = jnp.concatenate([dom_edge_index[1], pad_dst]).reshape(_ER, 128)
    dbatch2d = jnp.pad(dom_batch, (0, _NP - _N),
                       constant_values=64).reshape(_NR, 128)
    lbatch2d = jnp.pad(logs_batch, (0, _NP - _N),
                       constant_values=64).reshape(_NR, 128)

    wg1p = jnp.pad(Wg1, ((0, 14), (0, 0)))
    wt1p = jnp.pad(Wt1, ((0, 14), (0, 0)))
    w2c = jnp.zeros((128, 64), _F32).at[0:64, 0:32].set(Wg2).at[64:128, 32:64].set(Wt2)
    wl1c = jnp.concatenate([Wlg1, Wlt1], axis=1)
    bl1c = jnp.concatenate([blg1, blt1]).reshape(1, 50)
    w2bd = jnp.zeros((50, 64), _F32).at[0:25, 0:32].set(Wlg2).at[25:50, 32:64].set(Wlt2)

    degp = _deg_kernel(dst2d)
    dinvw = _dinv_tc(degp)
    y1p, mlp = _pre_tc(dom_xp, logs_xp, dinvw, wl1c, bl1c, w2bd)
    agg1 = _agg_kernel(src2d, dst2d, y1p)
    y2 = _mid_tc(agg1, y1p, dinvw, wg1p, bg1.reshape(1, 64), wt1p,
                 bt1.reshape(1, 64), w2c)
    agg2 = _agg_kernel(src2d, dst2d, y2)
    dpool = _pool_scaled(agg2, y2, dinvw, dbatch2d)
    lpool = _pool_plain(mlp, mlp, dinvw, lbatch2d)
    out = _final_tc(dpool, lpool, bg2.reshape(1, 32), bt2.reshape(1, 32),
                    blg2.reshape(1, 32), blt2.reshape(1, 32))
    return out.reshape(64)


# speedup vs baseline: 1.1715x; 1.1715x over previous
import functools

import jax
import jax.numpy as jnp
from jax import lax
from jax.experimental import pallas as pl
from jax.experimental.pallas import tpu as pltpu
from jax.experimental.pallas import tpu_sc as plsc

_N = 100000
_E = 1600000
_NP = 100352
_NR = 784
_EP = 1605632
_ER = 12544
_RANGE = 25088
_ACC_ROWS = 25216
_W = 64
_F32 = jnp.float32
_I32 = jnp.int32

_MESH = plsc.VectorSubcoreMesh(core_axis_name="c", subcore_axis_name="s")
_HIGH = lax.Precision.HIGHEST


def _mm(a, b):
    return jnp.dot(a, b, precision=_HIGH)


def _deg_body(dst2d, degp, dspm, dvw, zb, ones_v):
    c = lax.axis_index("c")
    t = lax.axis_index("s")

    def zf(i, _):
        zb[pl.ds(i * 16, 16)] = jnp.zeros((16,), _F32)
        return 0
    lax.fori_loop(0, 392, zf, 0)

    def of(i, _):
        ones_v[pl.ds(i * 16, 16)] = jnp.ones((16,), _F32)
        return 0
    lax.fori_loop(0, 8, of, 0)

    pltpu.sync_copy(zb, dspm.at[pl.ds(t * 6272, 6272)])
    plsc.subcore_barrier()

    row0 = (c * 16 + t) * 392

    def wbody(w, _):
        pltpu.sync_copy(dst2d.at[pl.ds(row0 + w * 8, 8)], dvw)
        for j in range(8):
            pltpu.sync_copy(ones_v.at[pl.ds(0, 128)], dspm.at[dvw.at[j]], add=True)
        return 0
    lax.fori_loop(0, 49, wbody, 0)
    plsc.subcore_barrier()

    pltpu.sync_copy(dspm.at[pl.ds(t * 6272, 6272)], zb.at[pl.ds(0, 6272)])
    pltpu.sync_copy(zb.at[pl.ds(0, 6272)], degp.at[c, pl.ds(t * 6272, 6272)])


_deg_kernel = functools.partial(
    pl.kernel,
    out_type=jax.ShapeDtypeStruct((2, _NP), _F32),
    mesh=_MESH,
    scratch_types=[
        pltpu.VMEM_SHARED((_NP,), _F32),
        pltpu.VMEM((8, 128), _I32),
        pltpu.VMEM((6272,), _F32),
        pltpu.VMEM((128,), _F32),
    ],
)(_deg_body)


def _agg_body(src2d, dst2d, y_hbm, agg, acc, svwA, dvwA, svwB, dvwB, fsrc,
              fldst, rows0, rows1, semA, semB, sem0, sem1):
    c = lax.axis_index("c")
    t = lax.axis_index("s")

    lane = lax.broadcasted_iota(_I32, (16,), 0)

    def _issue(g):
        r = g & 15
        b = g & 1
        for bi, (rw, sm) in enumerate(((rows0, sem0), (rows1, sem1))):
            @pl.when(b == bi)
            def _(rw=rw, sm=sm):
                pltpu.async_copy(y_hbm.at[fsrc.at[r]], rw, sm)

    def _wait_scatter(d):
        r = d & 15
        b = d & 1
        for bi, (rw, sm) in enumerate(((rows0, sem0), (rows1, sem1))):
            @pl.when(b == bi)
            def _(rw=rw, sm=sm):
                pltpu.make_async_copy(y_hbm.at[fsrc.at[r]], rw, sm).wait()
                pltpu.sync_copy(rw, acc.at[fldst.at[r]], add=True)

    for p in range(2):
        base = (2 * p + c) * _RANGE

        def zf(i, _):
            r = i // 4
            k = i % 4
            rows0[r, pl.ds(k * 16, 16)] = jnp.zeros((16,), _F32)
            return 0
        lax.fori_loop(0, 512, zf, 0)

        def zacc(k, _):
            blk = t + 16 * k

            @pl.when(blk < 197)
            def _():
                pltpu.sync_copy(rows0, acc.at[pl.ds(blk * 128, 128)])
            return 0
        lax.fori_loop(0, 13, zacc, 0)
        plsc.subcore_barrier()

        def _filter(svw, dvw, carry):
            cnt, gih, drn = carry
            for j in range(64):
                jr, jc = j // 8, (j % 8) * 16
                sv = svw[jr, pl.ds(jc, 16)]
                dv = dvw[jr, pl.ds(jc, 16)]
                m = (dv >= base) & (dv < base + _RANGE)
                mi = m.astype(_I32)
                incl = plsc.cumsum(mi)
                pos = cnt + incl - mi
                ridx = lax.shift_right_logical(pos, 7) & 15
                cidx = pos & 127
                plsc.store_scatter(fsrc, [ridx, cidx], sv, mask=m)
                plsc.store_scatter(fldst, [ridx, cidx], dv - base, mask=m)
                cnt = cnt + incl[15]

            def pcond(st):
                cnt_, gih_, drn_ = st
                return (gih_ + 1) * 128 <= cnt_

            def pbody(st):
                cnt_, gih_, drn_ = st
                full = gih_ - drn_ >= 2

                @pl.when(full)
                def _():
                    _wait_scatter(drn_)
                drn_ = jnp.where(full, drn_ + 1, drn_)
                _issue(gih_)
                return (cnt_, gih_ + 1, drn_)

            return lax.while_loop(pcond, pbody, (cnt, gih, drn))

        def _issue_win(w, sv_b, dv_b, sm):
            row0 = t * 784 + w * 8
            pltpu.async_copy(src2d.at[pl.ds(row0, 8)], sv_b, sm)
            pltpu.async_copy(dst2d.at[pl.ds(row0, 8)], dv_b, sm)

        def _wait_win(w, sv_b, dv_b, sm):
            row0 = t * 784 + w * 8
            pltpu.make_async_copy(src2d.at[pl.ds(row0, 8)], sv_b, sm).wait()
            pltpu.make_async_copy(dst2d.at[pl.ds(row0, 8)], dv_b, sm).wait()

        _issue_win(0, svwA, dvwA, semA)

        def wbody(i, carry):
            w = 2 * i
            _wait_win(w, svwA, dvwA, semA)
            _issue_win(w + 1, svwB, dvwB, semB)
            carry = _filter(svwA, dvwA, carry)
            _wait_win(w + 1, svwB, dvwB, semB)

            @pl.when(i < 48)
            def _():
                _issue_win(w + 2, svwA, dvwA, semA)
            carry = _filter(svwB, dvwB, carry)
            return carry

        cnt, gih, drn = lax.fori_loop(
            0, 49, wbody, (jnp.int32(0), jnp.int32(0), jnp.int32(0)))

        rem = cnt & 127

        @pl.when(rem > 0)
        def _():
            r = lax.shift_right_logical(cnt, 7) & 15
            rv = jnp.broadcast_to(r, (16,))
            for k in range(8):
                p128 = lane + k * 16
                mpad = p128 >= rem
                plsc.store_scatter(fsrc, [rv, p128], t * 128 + p128, mask=mpad)
                plsc.store_scatter(fldst, [rv, p128], _RANGE + p128, mask=mpad)

        def fcond(st):
            gih_, drn_ = st
            return drn_ < gih_

        def fbody(st):
            gih_, drn_ = st
            _wait_scatter(drn_)
            return (gih_, drn_ + 1)

        gih, drn = lax.while_loop(fcond, fbody, (gih, drn))

        @pl.when(rem > 0)
        def _():
            g = lax.shift_right_logical(cnt, 7)
            _issue(g)
            _wait_scatter(g)

        plsc.subcore_barrier()
        pltpu.sync_copy(acc.at[pl.ds(t * 1568, 1568)],
                        agg.at[pl.ds(base + t * 1568, 1568)])
        plsc.subcore_barrier()


_agg_kernel = functools.partial(
    pl.kernel,
    out_type=jax.ShapeDtypeStruct((_NP, _W), _F32),
    mesh=_MESH,
    compiler_params=pltpu.CompilerParams(needs_layout_passes=False, use_tc_tiling_on_sc=False),
    scratch_types=[
        pltpu.VMEM_SHARED((_ACC_ROWS, _W), _F32),
        pltpu.VMEM((8, 128), _I32),
        pltpu.VMEM((8, 128), _I32),
        pltpu.VMEM((8, 128), _I32),
        pltpu.VMEM((8, 128), _I32),
        pltpu.VMEM((16, 128), _I32),
        pltpu.VMEM((16, 128), _I32),
        pltpu.VMEM((128, _W), _F32),
        pltpu.VMEM((128, _W), _F32),
        pltpu.SemaphoreType.DMA,
        pltpu.SemaphoreType.DMA,
        pltpu.SemaphoreType.DMA,
        pltpu.SemaphoreType.DMA,
    ],
)(_agg_body)


def _pool_body(scaled, a_hbm, y_hbm, dinvw, batch2d, pooled, accp, av, yv,
               dvv, bvv):
    c = lax.axis_index("c")
    t = lax.axis_index("s")
    w = c * 16 + t

    def inf_f(i, _):
        r = i // 4
        k = i % 4
        accp[r, pl.ds(k * 16, 16)] = jnp.full((16,), -jnp.inf, _F32)
        return 0
    lax.fori_loop(0, 260, inf_f, 0)

    def gbody(gi, _):
        g = w + 32 * gi

        @pl.when(g < _NR)
        def _():
            pltpu.sync_copy(a_hbm.at[pl.ds(g * 128, 128)], av)
            if scaled:
                pltpu.sync_copy(y_hbm.at[pl.ds(g * 128, 128)], yv)
                pltpu.sync_copy(dinvw.at[g], dvv)
            pltpu.sync_copy(batch2d.at[g], bvv)

            def rbody(rc, _):
                b16 = bvv[pl.ds(rc * 16, 16)]
                if scaled:
                    d16 = dvv[pl.ds(rc * 16, 16)]
                for k16 in range(16):
                    r = rc * 16 + k16
                    b = b16[k16]
                    for k in range(4):
                        v = av[r, pl.ds(k * 16, 16)]
                        if scaled:
                            v = (v + yv[r, pl.ds(k * 16, 16)]) * d16[k16]
                        accp[b, pl.ds(k * 16, 16)] = jnp.maximum(
                            accp[b, pl.ds(k * 16, 16)], v)
                return 0
            lax.fori_loop(0, 8, rbody, 0)
        return 0
    lax.fori_loop(0, 25, gbody, 0)

    pltpu.sync_copy(accp, pooled.at[w])


def _make_pool(scaled):
    scratch = [
        pltpu.VMEM((65, _W), _F32),
        pltpu.VMEM((128, _W), _F32),
        pltpu.VMEM((128, _W), _F32),
        pltpu.VMEM((128,), _F32),
        pltpu.VMEM((128,), _I32),
    ]
    return functools.partial(
        pl.kernel,
        out_type=jax.ShapeDtypeStruct((32, 65, _W), _F32),
        mesh=_MESH,
        compiler_params=pltpu.CompilerParams(use_tc_tiling_on_sc=False),
        scratch_types=scratch,
    )(functools.partial(_pool_body, scaled))


_pool_scaled = _make_pool(True)
_pool_plain = _make_pool(False)


def _dinv_body(degp_ref, dinv_ref):
    s = degp_ref[0:1, :] + degp_ref[1:2, :] + 1.0
    dinv_ref[...] = lax.rsqrt(s).reshape(_NR, 128)


def _dinv_tc(degp):
    return pl.pallas_call(
        _dinv_body,
        out_shape=jax.ShapeDtypeStruct((_NR, 128), _F32),
    )(degp)


def _pre_body(domx_ref, logsx_ref, dinv_ref, wl1_ref, bl1_ref, wl2_ref,
              y1_ref, mlp_ref):
    di = dinv_ref[...].reshape(8, 128, 1)
    x3 = domx_ref[...].reshape(8, 128, 50)
    y1_ref[:, 0:50] = (x3 * di).reshape(1024, 50)
    y1_ref[:, 50:64] = jnp.zeros((1024, 14), _F32)
    h = jnp.maximum(_mm(logsx_ref[...], wl1_ref[...]) + bl1_ref[...], 0.0)
    mlp_ref[...] = _mm(h, wl2_ref[...])


def _pre_tc(domx, logsx, dinvw, wl1, bl1, wl2):
    nblk = _NP // 1024
    return pl.pallas_call(
        _pre_body,
        grid=(nblk,),
        in_specs=[
            pl.BlockSpec((1024, 50), lambda i: (i, 0)),
            pl.BlockSpec((1024, 50), lambda i: (i, 0)),
            pl.BlockSpec((8, 128), lambda i: (i, 0)),
            pl.BlockSpec((50, 50), lambda i: (0, 0)),
            pl.BlockSpec((1, 50), lambda i: (0, 0)),
            pl.BlockSpec((50, 64), lambda i: (0, 0)),
        ],
        out_specs=[
            pl.BlockSpec((1024, 64), lambda i: (i, 0)),
            pl.BlockSpec((1024, 64), lambda i: (i, 0)),
        ],
        out_shape=[
            jax.ShapeDtypeStruct((_NP, 64), _F32),
            jax.ShapeDtypeStruct((_NP, 64), _F32),
        ],
    )(domx, logsx, dinvw, wl1, bl1, wl2)


def _mid_body(agg1_ref, y1_ref, dinv_ref, wg1_ref, bg1_ref, wt1_ref, bt1_ref,
              w2c_ref, y2_ref):
    di = dinv_ref[...].reshape(8, 128, 1)
    z1 = ((agg1_ref[...] + y1_ref[...]).reshape(8, 128, 64) * di).reshape(1024, 64)
    hg = jnp.maximum(_mm(z1, wg1_ref[...]) + bg1_ref[...], 0.0)
    ht = jnp.maximum(_mm(z1, wt1_ref[...]) + bt1_ref[...], 0.0)
    q = _mm(jnp.concatenate([hg, ht], axis=1), w2c_ref[...])
    y2_ref[...] = (q.reshape(8, 128, 64) * di).reshape(1024, 64)


def _mid_tc(agg1, y1p, dinvw, wg1, bg1, wt1, bt1, w2c):
    nblk = _NP // 1024
    return pl.pallas_call(
        _mid_body,
        grid=(nblk,),
        in_specs=[
            pl.BlockSpec((1024, 64), lambda i: (i, 0)),
            pl.BlockSpec((1024, 64), lambda i: (i, 0)),
            pl.BlockSpec((8, 128), lambda i: (i, 0)),
            pl.BlockSpec((64, 64), lambda i: (0, 0)),
            pl.BlockSpec((1, 64), lambda i: (0, 0)),
            pl.BlockSpec((64, 64), lambda i: (0, 0)),
            pl.BlockSpec((1, 64), lambda i: (0, 0)),
            pl.BlockSpec((128, 64), lambda i: (0, 0)),
        ],
        out_specs=pl.BlockSpec((1024, 64), lambda i: (i, 0)),
        out_shape=jax.ShapeDtypeStruct((_NP, 64), _F32),
    )(agg1, y1p, dinvw, wg1, bg1, wt1, bt1, w2c)


def _final_body(dp_ref, lp_ref, bg2_ref, bt2_ref, blg2_ref, blt2_ref, out_ref):
    dp = jnp.max(dp_ref[...], axis=0)
    lp = jnp.max(lp_ref[...], axis=0)
    gd = dp[0:64, 0:32] + bg2_ref[...]
    ad = dp[0:64, 32:64] + bt2_ref[...]
    gl = lp[0:64, 0:32] + blg2_ref[...]
    al = lp[0:64, 32:64] + blt2_ref[...]
    d1 = ad - gd + 1e-6
    d2 = al - gl + 1e-6
    out_ref[...] = (jnp.sqrt(jnp.sum(d1 * d1, axis=1, keepdims=True))
                    + jnp.sqrt(jnp.sum(d2 * d2, axis=1, keepdims=True)))


def _final_tc(dpool, lpool, bg2, bt2, blg2, blt2):
    return pl.pallas_call(
        _final_body,
        out_shape=jax.ShapeDtypeStruct((64, 1), _F32),
    )(dpool, lpool, bg2, bt2, blg2, blt2)


def kernel(dom_x, dom_edge_index, dom_batch, logs_x, logs_batch, Wg1, bg1, Wg2,
           bg2, Wt1, bt1, Wt2, bt2, Wlg1, blg1, Wlg2, blg2, Wlt1, blt1, Wlt2,
           blt2):
    dom_xp = jnp.pad(dom_x, ((0, _NP - _N), (0, 0)))
    logs_xp = jnp.pad(logs_x, ((0, _NP - _N), (0, 0)))
    npad = _EP - _E
    pidx = lax.iota(_I32, npad)
    pad_src = (pidx * 17) % _N
    pad_dst = _N + (pidx % (_NP - _N))
    src2d = jnp.concatenate([dom_edge_index[0], pad_src]).reshape(_ER, 128)
    dst2d = jnp.concatenate([dom_edge_index[1], pad_dst]).reshape(_ER, 128)
    dbatch2d = jnp.pad(dom_batch, (0, _NP - _N),
                       constant_values=64).reshape(_NR, 128)
    lbatch2d = jnp.pad(logs_batch, (0, _NP - _N),
                       constant_values=64).reshape(_NR, 128)

    wg1p = jnp.pad(Wg1, ((0, 14), (0, 0)))
    wt1p = jnp.pad(Wt1, ((0, 14), (0, 0)))
    w2c = jnp.zeros((128, 64), _F32).at[0:64, 0:32].set(Wg2).at[64:128, 32:64].set(Wt2)
    wl1c = jnp.concatenate([Wlg1, Wlt1], axis=1)
    bl1c = jnp.concatenate([blg1, blt1]).reshape(1, 50)
    w2bd = jnp.zeros((50, 64), _F32).at[0:25, 0:32].set(Wlg2).at[25:50, 32:64].set(Wlt2)

    degp = _deg_kernel(dst2d)
    dinvw = _dinv_tc(degp)
    y1p, mlp = _pre_tc(dom_xp, logs_xp, dinvw, wl1c, bl1c, w2bd)
    agg1 = _agg_kernel(src2d, dst2d, y1p)
    y2 = _mid_tc(agg1, y1p, dinvw, wg1p, bg1.reshape(1, 64), wt1p,
                 bt1.reshape(1, 64), w2c)
    agg2 = _agg_kernel(src2d, dst2d, y2)
    dpool = _pool_scaled(agg2, y2, dinvw, dbatch2d)
    lpool = _pool_plain(mlp, mlp, dinvw, lbatch2d)
    out = _final_tc(dpool, lpool, bg2.reshape(1, 32), bt2.reshape(1, 32),
                    blg2.reshape(1, 32), blt2.reshape(1, 32))
    return out.reshape(64)
